# Initial kernel scaffold; baseline (speedup 1.0000x reference)
import jax
import jax.numpy as jnp
from jax.experimental import pallas as pl


def kernel(x, edge_index, params):
    raise NotImplementedError("write your pallas kernel here")


# speedup vs baseline: 35.1956x; 35.1956x over previous
import functools
import jax
import jax.numpy as jnp
from jax import lax
from jax.experimental import pallas as pl
from jax.experimental.pallas import tpu as pltpu
from jax.experimental.pallas import tpu_sc as plsc

N = 10000
E = 320000
D_IN = 128
HEADS = 8
HID = 16
TH = 128
OUT = 40

NP = 10240
ND = N
NC = 2
NS = 16
NW = NC * NS
C = 128
RPT = NP // NS

ESL = E + N
EPW1 = -(-ESL // (NW * C)) * C
EP1 = EPW1 * NW
NCH1 = EPW1 // C
EPW2 = -(-E // (NW * C)) * C
EP2 = EPW2 * NW
NCH2 = EPW2 // C

_HI = jax.lax.Precision.HIGHEST


def _dot(a, b):
    return jax.lax.dot_general(a, b, (((1,), (0,)), ((), ())),
                               precision=_HI, preferred_element_type=jnp.float32)


def _k1_body(x_ref, rw_ref, rb_ref, gw_ref, ac_ref,
             xp_ref, xw_ref, s2_ref, d2_ref):
    xp = _dot(x_ref[...], rw_ref[...]) + rb_ref[...]
    xw = _dot(xp, gw_ref[...])
    sd = _dot(xw, ac_ref[...])
    xp_ref[...] = xp
    xw_ref[...] = xw
    s2_ref[...] = sd[:, :16]
    d2_ref[...] = sd[:, 16:]


def _k2_body(op_ref, dp_ref, em_ref, gb_ref, s_ref, t_ref, gw_ref, ac_ref,
             xw_ref, s2_ref, d2_ref):
    un = op_ref[0] + op_ref[1]
    den = dp_ref[0] + dp_ref[1]
    dex = _dot(den, em_ref[...])
    g = un / (dex + 1e-16) + gb_ref[...]
    g = g * s_ref[...] + t_ref[...]
    h = jnp.where(g > 0, g, jnp.exp(g) - 1.0)
    xw = _dot(h, gw_ref[...])
    sd = _dot(xw, ac_ref[...])
    xw_ref[...] = xw
    s2_ref[...] = sd[:, :16]
    d2_ref[...] = sd[:, 16:]


def _k3_body(op_ref, dp_ref, em_ref, gb_ref, s_ref, t_ref, wa_ref, wb_ref,
             b1_ref, hb_ref, a_ref, b_ref):
    un = op_ref[0] + op_ref[1]
    den = dp_ref[0] + dp_ref[1]
    dex = _dot(den, em_ref[...])
    g = un / (dex + 1e-16) + gb_ref[...]
    g = g * s_ref[...] + t_ref[...]
    h = jnp.where(g > 0, g, jnp.exp(g) - 1.0)
    hb_ref[...] = h
    a_ref[...] = _dot(h, wa_ref[...])
    b_ref[...] = _dot(h, wb_ref[...]) + b1_ref[...]


def _k4_body(hb_ref, ag_ref, w1_ref, b1_ref, s_ref, t_ref, w2_ref, b2_ref,
             out_ref):
    hs = hb_ref[...] + ag_ref[0] + ag_ref[1]
    c1 = _dot(hs, w1_ref[...]) + b1_ref[...]
    c1 = c1 * s_ref[...] + t_ref[...]
    c1 = jnp.maximum(c1, 0.0)
    lg = _dot(c1, w2_ref[...]) + b2_ref[...]
    m = jnp.max(lg, axis=1, keepdims=True)
    lse = m + jnp.log(jnp.sum(jnp.exp(lg - m), axis=1, keepdims=True))
    out_ref[...] = lg - lse


def _row_spec(rb, cols):
    return pl.BlockSpec((rb, cols), lambda i: (i, 0))


def _full_spec(shape):
    nd = len(shape)
    return pl.BlockSpec(shape, lambda i: (0,) * nd)


_RB = 1024
_GRID = NP // _RB


def _tc_call(body, in_specs, out_specs, out_shapes, args):
    return pl.pallas_call(
        body,
        grid=(_GRID,),
        in_specs=in_specs,
        out_specs=out_specs,
        out_shape=out_shapes,
    )(*args)


_MESH = plsc.VectorSubcoreMesh(core_axis_name="c", subcore_axis_name="s")
_SC_PARAMS = pltpu.CompilerParams(use_tc_tiling_on_sc=False,
                                  needs_layout_passes=False)


def _gat_edge_body(rowsl, colsl, src2, dst2, xw, outp, denp,
                   ridx, cidx, srow, dcol, xwr, pbuf, out_sp, den_sp,
                   sem1, sem2, sem3):
    cid = lax.axis_index("c")
    sid = lax.axis_index("s")
    w = cid * NS + sid

    zrow = sid * RPT
    def _z16(i, _):
        srow[i] = jnp.zeros((16,), jnp.float32)
        return 0
    lax.fori_loop(0, C, _z16, 0)
    def _zstripe16(i, _):
        pltpu.sync_copy(srow, den_sp.at[pl.ds(zrow + i * C, C)])
        return 0
    lax.fori_loop(0, RPT // C, _zstripe16, 0)
    def _z128(i, _):
        for k in range(8):
            xwr[i, pl.ds(k * 16, 16)] = jnp.zeros((16,), jnp.float32)
        return 0
    lax.fori_loop(0, C, _z128, 0)
    def _zstripe128(i, _):
        pltpu.sync_copy(xwr, out_sp.at[pl.ds(zrow + i * C, C)])
        return 0
    lax.fori_loop(0, RPT // C, _zstripe128, 0)
    plsc.subcore_barrier()

    def chunk(ch, _):
        base = w * EPW1 + ch * C
        pltpu.sync_copy(rowsl.at[pl.ds(base, C)], ridx)
        pltpu.sync_copy(colsl.at[pl.ds(base, C)], cidx)
        d1 = pltpu.async_copy(src2.at[ridx], srow, sem1)
        d2 = pltpu.async_copy(dst2.at[cidx], dcol, sem2)
        d3 = pltpu.async_copy(xw.at[ridx], xwr, sem3)
        d1.wait()
        d2.wait()
        d3.wait()

        def edge(i, _2):
            a = srow[i] + dcol[i]
            lr = jnp.maximum(a, a * 0.2)
            pe = jnp.exp(lr)
            pbuf[i] = pe
            for h in range(HEADS):
                ph = pe[h]
                blk = xwr[i, pl.ds(h * 16, 16)]
                xwr[i, pl.ds(h * 16, 16)] = blk * ph
            return 0
        lax.fori_loop(0, C, edge, 0)

        pltpu.sync_copy(pbuf, den_sp.at[cidx], add=True)
        pltpu.sync_copy(xwr, out_sp.at[cidx], add=True)
        return 0
    lax.fori_loop(0, NCH1, chunk, 0)
    plsc.subcore_barrier()

    pltpu.sync_copy(out_sp.at[pl.ds(zrow, RPT)], outp.at[cid, pl.ds(zrow, RPT)])
    pltpu.sync_copy(den_sp.at[pl.ds(zrow, RPT)], denp.at[cid, pl.ds(zrow, RPT)])


def _gat_edge_pass(rowsl, colsl, src2, dst2, xw):
    f = pl.kernel(
        _gat_edge_body,
        out_type=(
            jax.ShapeDtypeStruct((NC, NP, TH), jnp.float32),
            jax.ShapeDtypeStruct((NC, NP, 16), jnp.float32),
        ),
        mesh=_MESH,
        scratch_types=(
            pltpu.VMEM((C,), jnp.int32),
            pltpu.VMEM((C,), jnp.int32),
            pltpu.VMEM((C, 16), jnp.float32),
            pltpu.VMEM((C, 16), jnp.float32),
            pltpu.VMEM((C, TH), jnp.float32),
            pltpu.VMEM((C, 16), jnp.float32),
            pltpu.VMEM_SHARED((NP, TH), jnp.float32),
            pltpu.VMEM_SHARED((NP, 16), jnp.float32),
            pltpu.SemaphoreType.DMA,
            pltpu.SemaphoreType.DMA,
            pltpu.SemaphoreType.DMA,
        ),
        compiler_params=_SC_PARAMS,
    )
    return f(rowsl, colsl, src2, dst2, xw)


def _kernel_impl(x, edge_index, params, consts):
    p = params
    row = edge_index[0]
    col = edge_index[1]
    (a1cat, a2cat, em, s1, t1, s2, t2, cs, ct, w2p, b2p, tfix) = consts

    sl = jnp.arange(N, dtype=jnp.int32)
    padE1 = jnp.full((EP1 - ESL,), ND, jnp.int32)
    rowsl = jnp.concatenate([row, sl, padE1])
    colsl = jnp.concatenate([col, sl, padE1])
    padE2 = jnp.full((EP2 - E,), ND, jnp.int32)
    rowp = jnp.concatenate([row, padE2])
    colp = jnp.concatenate([col, padE2])
    tpad = jnp.concatenate([tfix, jnp.full((EP2 - E,), 1e30, jnp.float32)])

    xpad = jnp.zeros((NP, D_IN), jnp.float32).at[:N].set(x)

    rb2 = p["res_b"].reshape(1, TH)
    g1b = p["g1_b"].reshape(1, TH)
    g2b = p["g2_b"].reshape(1, TH)

    xp, xw1, s2a, d2a = _tc_call(
        _k1_body,
        [_row_spec(_RB, D_IN), _full_spec((D_IN, TH)), _full_spec((1, TH)),
         _full_spec((TH, TH)), _full_spec((TH, 32))],
        [_row_spec(_RB, TH), _row_spec(_RB, TH), _row_spec(_RB, 16), _row_spec(_RB, 16)],
        [jax.ShapeDtypeStruct((NP, TH), jnp.float32),
         jax.ShapeDtypeStruct((NP, TH), jnp.float32),
         jax.ShapeDtypeStruct((NP, 16), jnp.float32),
         jax.ShapeDtypeStruct((NP, 16), jnp.float32)],
        [xpad, p["res_W"], rb2, p["g1_W"], a1cat],
    )

    outp1, denp1 = _gat_edge_pass(rowsl, colsl, s2a, d2a, xw1)

    xw2, s2b, d2b = _tc_call(
        _k2_body,
        [pl.BlockSpec((NC, _RB, TH), lambda i: (0, i, 0)),
         pl.BlockSpec((NC, _RB, 16), lambda i: (0, i, 0)),
         _full_spec((16, TH)), _full_spec((1, TH)), _full_spec((1, TH)),
         _full_spec((1, TH)), _full_spec((TH, TH)), _full_spec((TH, 32))],
        [_row_spec(_RB, TH), _row_spec(_RB, 16), _row_spec(_RB, 16)],
        [jax.ShapeDtypeStruct((NP, TH), jnp.float32),
         jax.ShapeDtypeStruct((NP, 16), jnp.float32),
         jax.ShapeDtypeStruct((NP, 16), jnp.float32)],
        [outp1, denp1, em, g1b, s1.reshape(1, TH), t1.reshape(1, TH),
         p["g2_W"], a2cat],
    )

    outp2, denp2 = _gat_edge_pass(rowsl, colsl, s2b, d2b, xw2)

    hb, abuf, bbuf = _tc_call(
        _k3_body,
        [pl.BlockSpec((NC, _RB, TH), lambda i: (0, i, 0)),
         pl.BlockSpec((NC, _RB, 16), lambda i: (0, i, 0)),
         _full_spec((16, TH)), _full_spec((1, TH)), _full_spec((1, TH)),
         _full_spec((1, TH)), _full_spec((TH, 64)), _full_spec((TH, 64)),
         _full_spec((1, 64))],
        [_row_spec(_RB, TH), _row_spec(_RB, 64), _row_spec(_RB, 64)],
        [jax.ShapeDtypeStruct((NP, TH), jnp.float32),
         jax.ShapeDtypeStruct((NP, 64), jnp.float32),
         jax.ShapeDtypeStruct((NP, 64), jnp.float32)],
        [outp2, denp2, em, g2b, s2.reshape(1, TH), t2.reshape(1, TH),
         p["s_W1"][:TH], p["s_W1"][TH:], p["s_b1"].reshape(1, 64)],
    )

    w2st = jnp.tile(p["s_W2"][:, :1], (1, 16))
    sb2v = jnp.full((16,), p["s_b2"][0], jnp.float32)
    lg_pad, w_pad, aggp = _scorer_pass(rowp, colp, tpad, abuf, bbuf, hb, w2st, sb2v)

    (out,) = _tc_call(
        _k4_body,
        [_row_spec(_RB, TH),
         pl.BlockSpec((NC, _RB, TH), lambda i: (0, i, 0)),
         _full_spec((TH, 64)), _full_spec((1, 64)), _full_spec((1, 64)),
         _full_spec((1, 64)), _full_spec((64, TH)), _full_spec((1, TH))],
        [_row_spec(_RB, TH)],
        [jax.ShapeDtypeStruct((NP, TH), jnp.float32)],
        [hb, aggp, p["c_W1"], p["c_b1"].reshape(1, 64), cs.reshape(1, 64),
         ct.reshape(1, 64), w2p, b2p.reshape(1, TH)],
    )

    return out[:N, :OUT], w_pad[:E], lg_pad[:E]


def _scorer_body2(rowp, colp, tpad, abuf_h, bbuf_h, hb_h, w2s_h, sb2v_h,
                  lg_out, w_out, aggp,
                  ridx, cidx, tbuf, arow, bcol, hcol, lbuf, wbuf, sidx, w2s,
                  sb2b, agg_sp, sem1, sem2, sem3):
    cid = lax.axis_index("c")
    sid = lax.axis_index("s")
    w = cid * NS + sid
    zrow = sid * RPT

    def _z128(i, _):
        for k in range(8):
            hcol[i, pl.ds(k * 16, 16)] = jnp.zeros((16,), jnp.float32)
        return 0
    lax.fori_loop(0, C, _z128, 0)
    def _zstripe(i, _):
        pltpu.sync_copy(hcol, agg_sp.at[pl.ds(zrow + i * C, C)])
        return 0
    lax.fori_loop(0, RPT // C, _zstripe, 0)
    plsc.subcore_barrier()

    pltpu.sync_copy(w2s_h, w2s)
    pltpu.sync_copy(sb2v_h, sb2b)

    def chunk(ch, _):
        base = w * EPW2 + ch * C
        pltpu.sync_copy(rowp.at[pl.ds(base, C)], ridx)
        pltpu.sync_copy(colp.at[pl.ds(base, C)], cidx)
        pltpu.sync_copy(tpad.at[pl.ds(base, C)], tbuf)
        d1 = pltpu.async_copy(abuf_h.at[ridx], arow, sem1)
        d2 = pltpu.async_copy(bbuf_h.at[cidx], bcol, sem2)
        d3 = pltpu.async_copy(hb_h.at[cidx], hcol, sem3)
        d1.wait()
        d2.wait()
        d3.wait()

        iota = lax.iota(jnp.int32, 16)

        def group(j, _2):
            base16 = j * 16
            eidx = base16 + iota
            acc = sb2b[...]
            for k in range(64):
                kv = jnp.full((16,), k, jnp.int32)
                av = plsc.load_gather(arow, [eidx, kv])
                bv = plsc.load_gather(bcol, [eidx, kv])
                v = jnp.maximum(av + bv, 0.0)
                acc = acc + v * w2s[k]
            tv = tbuf[pl.ds(base16, 16)]
            rv = ridx[pl.ds(base16, 16)]
            m = acc > tv
            lbuf[pl.ds(base16, 16)] = acc
            wbuf[pl.ds(base16, 16)] = jnp.where(m, 1.0, 0.0)
            sidx[pl.ds(base16, 16)] = jnp.where(m, rv, ND)
            return 0
        lax.fori_loop(0, C // 16, group, 0)

        pltpu.sync_copy(lbuf, lg_out.at[pl.ds(base, C)])
        pltpu.sync_copy(wbuf, w_out.at[pl.ds(base, C)])
        pltpu.sync_copy(hcol, agg_sp.at[sidx], add=True)
        return 0
    lax.fori_loop(0, NCH2, chunk, 0)
    plsc.subcore_barrier()

    pltpu.sync_copy(agg_sp.at[pl.ds(zrow, RPT)], aggp.at[cid, pl.ds(zrow, RPT)])


def _scorer_pass(rowp, colp, tpad, abuf, bbuf, hb, w2st, sb2v):
    f = pl.kernel(
        _scorer_body2,
        out_type=(
            jax.ShapeDtypeStruct((EP2,), jnp.float32),
            jax.ShapeDtypeStruct((EP2,), jnp.float32),
            jax.ShapeDtypeStruct((NC, NP, TH), jnp.float32),
        ),
        mesh=_MESH,
        scratch_types=(
            pltpu.VMEM((C,), jnp.int32),
            pltpu.VMEM((C,), jnp.int32),
            pltpu.VMEM((C,), jnp.float32),
            pltpu.VMEM((C, 64), jnp.float32),
            pltpu.VMEM((C, 64), jnp.float32),
            pltpu.VMEM((C, TH), jnp.float32),
            pltpu.VMEM((C,), jnp.float32),
            pltpu.VMEM((C,), jnp.float32),
            pltpu.VMEM((C,), jnp.int32),
            pltpu.VMEM((64, 16), jnp.float32),
            pltpu.VMEM((16,), jnp.float32),
            pltpu.VMEM_SHARED((NP, TH), jnp.float32),
            pltpu.SemaphoreType.DMA,
            pltpu.SemaphoreType.DMA,
            pltpu.SemaphoreType.DMA,
        ),
        compiler_params=_SC_PARAMS,
    )
    return f(rowp, colp, tpad, abuf, bbuf, hb, w2st, sb2v)


def _make_consts(params):
    p = params

    def acat(a_s, a_d):
        eye = jnp.eye(HEADS, dtype=jnp.float32)
        ms = (a_s[:, :, None] * eye[:, None, :]).reshape(TH, HEADS)
        md = (a_d[:, :, None] * eye[:, None, :]).reshape(TH, HEADS)
        return jnp.concatenate([ms, ms, md, md], axis=1)

    a1cat = acat(p["g1_as"], p["g1_ad"])
    a2cat = acat(p["g2_as"], p["g2_ad"])
    em = jnp.concatenate(
        [jnp.kron(jnp.eye(HEADS, dtype=jnp.float32), jnp.ones((1, 16), jnp.float32)),
         jnp.zeros((8, TH), jnp.float32)], axis=0)

    def bnst(g, b, m, v):
        s = g / jnp.sqrt(v + 1e-5)
        return s, b - m * s

    s1, t1 = bnst(p["bn1_g"], p["bn1_b"], p["bn1_m"], p["bn1_v"])
    s2, t2 = bnst(p["bn2_g"], p["bn2_b"], p["bn2_m"], p["bn2_v"])
    cs, ct = bnst(p["cbn_g"], p["cbn_b"], p["cbn_m"], p["cbn_v"])

    w2p = jnp.zeros((64, TH), jnp.float32).at[:, :OUT].set(p["c_W2"])
    b2p = jnp.full((TH,), -1e30, jnp.float32).at[:OUT].set(p["c_b2"])

    u = jax.random.uniform(jax.random.key(42), (E, 2),
                           minval=1e-6, maxval=1.0 - 1e-6)
    g = -jnp.log(-jnp.log(u))
    tfix = g[:, 0] - g[:, 1]

    return (a1cat, a2cat, em, s1, t1, s2, t2, cs, ct, w2p, b2p, tfix)


@jax.jit
def kernel(x, edge_index, params):
    consts = _make_consts(params)
    return _kernel_impl(x, edge_index.astype(jnp.int32), params, consts)
